# Initial kernel scaffold; baseline (speedup 1.0000x reference)
import jax
import jax.numpy as jnp
from jax.experimental import pallas as pl


def kernel(batch, table, W, cls_w, cls_b):
    raise NotImplementedError("write your pallas kernel here")


# speedup vs baseline: 1.4168x; 1.4168x over previous
import functools

import jax
import jax.numpy as jnp
from jax import lax
from jax.experimental import pallas as pl
from jax.experimental.pallas import tpu as pltpu
from jax.experimental.pallas import tpu_sc as plsc

NC = 2
NS = 16
NW = NC * NS

B = 4096
D = 64
M = 16

_mesh = plsc.VectorSubcoreMesh(core_axis_name="c", subcore_axis_name="s")


NUM_IDX = 4 * B
G_PER_W = NUM_IDX // NW


@functools.partial(
    pl.kernel,
    mesh=_mesh,
    out_type=jax.ShapeDtypeStruct((NUM_IDX, D), jnp.float32),
    scratch_types=[
        pltpu.VMEM((G_PER_W,), jnp.int32),
        pltpu.VMEM((G_PER_W, D), jnp.float32),
        pltpu.SemaphoreType.DMA,
    ],
)
def _sc_gather(idx_hbm, table_hbm, out_hbm, idx_v, rows_v, sem):
    wid = lax.axis_index("s") * NC + lax.axis_index("c")
    base = wid * G_PER_W
    pltpu.sync_copy(idx_hbm.at[pl.ds(base, G_PER_W)], idx_v)

    def body(g, carry):
        vec = idx_v[pl.ds(g * 16, 16)]
        for j in range(16):
            r = vec[j]
            pltpu.async_copy(
                table_hbm.at[pl.ds(r, 1), :],
                rows_v.at[pl.ds(g * 16 + j, 1), :], sem)
        return carry

    lax.fori_loop(0, G_PER_W // 16, body, 0)
    pltpu.make_async_copy(
        out_hbm.at[pl.ds(base, G_PER_W)], rows_v, sem).wait()
    pltpu.sync_copy(rows_v, out_hbm.at[pl.ds(base, G_PER_W)])


S_PER_W = B // NW


@functools.partial(
    pl.kernel,
    mesh=_mesh,
    out_type=(),
    scratch_types=[
        pltpu.VMEM((S_PER_W,), jnp.int32),
        pltpu.VMEM((S_PER_W, D), jnp.float32),
        pltpu.SemaphoreType.DMA,
    ],
)
def _sc_scatter(idx_hbm, rows_hbm, table_ref, idx_v, rows_v, sem):
    wid = lax.axis_index("s") * NC + lax.axis_index("c")
    base = wid * S_PER_W
    pltpu.sync_copy(idx_hbm.at[pl.ds(base, S_PER_W)], idx_v)
    pltpu.sync_copy(rows_hbm.at[pl.ds(base, S_PER_W)], rows_v)

    def body(g, carry):
        vec = idx_v[pl.ds(g * 16, 16)]
        for j in range(16):
            r = vec[j]
            pltpu.async_copy(
                rows_v.at[pl.ds(g * 16 + j, 1), :],
                table_ref.at[pl.ds(r, 1), :], sem)
        return carry

    lax.fori_loop(0, S_PER_W // 16, body, 0)
    pltpu.make_async_copy(
        rows_hbm.at[pl.ds(base, S_PER_W)], rows_v, sem).wait()


BB = 512


def _tc_body(mods_ref, x0_ref, n1_ref, n2_ref, el_ref, w2_ref, clsw_ref,
             clsb_ref, rows_ref, logits_ref):
    f32 = jnp.float32
    mods = mods_ref[...]
    fold = jnp.where(
        lax.broadcasted_iota(jnp.int32, (M * D, D), 0) % D
        == lax.broadcasted_iota(jnp.int32, (M * D, D), 1),
        1.0, 0.0).astype(f32)
    group = lax.broadcasted_iota(jnp.int32, (BB, M * D), 1) // D
    w2 = w2_ref[...]
    x = x0_ref[...]
    for j, bias_ref in enumerate((n1_ref, n2_ref, el_ref)):
        yall = jnp.dot(x, w2, preferred_element_type=f32)
        mj = mods[:, j:j + 1]
        masked = jnp.where(group == mj, yall, 0.0)
        y = jnp.dot(masked, fold, preferred_element_type=f32) + bias_ref[...]
        x = jnp.maximum(y, 0.0) if j < 2 else y
    rows_ref[...] = 0.9 * el_ref[...] + 0.1 * x
    logits_ref[...] = (
        jnp.dot(x, clsw_ref[...], preferred_element_type=f32) + clsb_ref[...])


def _tc_compute(mods, x0, n1, n2, el, w2, clsw_t, clsb2):
    grid = (B // BB,)
    blk = lambda c: pl.BlockSpec((BB, c), lambda i: (i, 0))
    full = lambda r, c: pl.BlockSpec((r, c), lambda i: (0, 0))
    return pl.pallas_call(
        _tc_body,
        grid=grid,
        in_specs=[blk(3), blk(D), blk(D), blk(D), blk(D),
                  full(D, M * D), full(D, M), full(1, M)],
        out_specs=[blk(D), blk(M)],
        out_shape=[jax.ShapeDtypeStruct((B, D), jnp.float32),
                   jax.ShapeDtypeStruct((B, M), jnp.float32)],
    )(mods, x0, n1, n2, el, w2, clsw_t, clsb2)


def kernel(batch, table, W, cls_w, cls_b):
    idx_all = jnp.concatenate(
        [batch[:, 0], batch[:, 2], batch[:, 4], batch[:, 6]])
    rows = _sc_gather(idx_all, table)
    x0 = rows[0 * B:1 * B]
    n1 = rows[1 * B:2 * B]
    n2 = rows[2 * B:3 * B]
    el = rows[3 * B:4 * B]
    mods = batch[:, 1::2]
    w2 = jnp.transpose(W, (2, 0, 1)).reshape(D, M * D)
    out_rows, logits = _tc_compute(
        mods, x0, n1, n2, el, w2, cls_w.T, cls_b.reshape(1, M))
    tref = jax.new_ref(table)
    _sc_scatter(batch[:, 6], out_rows, tref)
    new_table = jax.freeze(tref)
    return logits, new_table
